# Initial kernel scaffold; baseline (speedup 1.0000x reference)
import jax
import jax.numpy as jnp
from jax.experimental import pallas as pl


def kernel(node_feat, edge_idx, edge_diff_embedding, edge_dist_embedding, node_attr, W1, Wm1, Wm2, Wm3, Wm4, W2, Wskip):
    raise NotImplementedError("write your pallas kernel here")


# speedup vs baseline: 2.2677x; 2.2677x over previous
import functools

import jax
import jax.numpy as jnp
from jax import lax
from jax.experimental import pallas as pl
from jax.experimental.pallas import tpu as pltpu
from jax.experimental.pallas import tpu_sc as plsc

N = 10000
E = 320000
D = 128
RB = 8
ZA = 4
HID = 64
AVG_INV = 1.0 / 32.0

NC = 2
NS = 16
NW = NC * NS
CHUNK = 128
CPT = 80
EPAD = CPT * CHUNK * NW
NPAD = 10240
NPS = NPAD // NS


def _edge_body(dist_ref, diff_ref, wm1_ref, wm2_ref, wm3_ref, v_ref):
    x = jnp.dot(dist_ref[...], wm1_ref[...], preferred_element_type=jnp.float32)
    x = x * jax.nn.sigmoid(x)
    x = jnp.dot(x, wm2_ref[...], preferred_element_type=jnp.float32)
    x = x * jax.nn.sigmoid(x)
    x = jnp.dot(x, wm3_ref[...], preferred_element_type=jnp.float32)
    x = x * jax.nn.sigmoid(x)
    v_ref[...] = x * diff_ref[...]


def _edge_stage(dist_pad, diff_pad, Wm1, Wm2, Wm3):
    BE = 2048
    return pl.pallas_call(
        _edge_body,
        grid=(EPAD // BE,),
        in_specs=[
            pl.BlockSpec((BE, RB), lambda i: (i, 0)),
            pl.BlockSpec((BE, 1), lambda i: (i, 0)),
            pl.BlockSpec((RB, HID), lambda i: (0, 0)),
            pl.BlockSpec((HID, HID), lambda i: (0, 0)),
            pl.BlockSpec((HID, HID), lambda i: (0, 0)),
        ],
        out_specs=pl.BlockSpec((BE, HID), lambda i: (i, 0)),
        out_shape=jax.ShapeDtypeStruct((EPAD, HID), jnp.float32),
    )(dist_pad, diff_pad, Wm1, Wm2, Wm3)


def _sc_body(v_hbm, idx_hbm, zeros_hbm, out_hbm, shared, idx_v, vbuf):
    cid = lax.axis_index("c")
    sid = lax.axis_index("s")
    wid = sid * NC + cid
    pltpu.sync_copy(zeros_hbm, shared.at[pl.ds(sid * NPS, NPS)])
    plsc.subcore_barrier()

    def step(j, carry):
        row = wid * CPT + j
        pltpu.sync_copy(idx_hbm.at[row], idx_v)
        pltpu.sync_copy(v_hbm.at[pl.ds(row * CHUNK, CHUNK)], vbuf)
        pltpu.sync_copy(vbuf, shared.at[idx_v], add=True)
        return carry

    lax.fori_loop(0, CPT, step, 0)
    plsc.subcore_barrier()
    out_base = cid * NPAD + sid * NPS
    pltpu.sync_copy(shared.at[pl.ds(sid * NPS, NPS)],
                    out_hbm.at[pl.ds(out_base, NPS)])


@functools.cache
def _get_sc_scatter():
    return pl.kernel(
        _sc_body,
        out_type=jax.ShapeDtypeStruct((NC * NPAD, HID), jnp.float32),
        mesh=plsc.VectorSubcoreMesh(core_axis_name="c", subcore_axis_name="s",
                                    num_cores=NC, num_subcores=NS),
        scratch_types=[
            pltpu.VMEM_SHARED((NPAD, HID), jnp.float32),
            pltpu.VMEM((CHUNK,), jnp.int32),
            pltpu.VMEM((CHUNK, HID), jnp.float32),
        ],
        compiler_params=pltpu.CompilerParams(use_tc_tiling_on_sc=False),
    )


def _node_body(nf_ref, p0_ref, p1_ref, attr_ref, w1_ref, wm4_ref, w2_ref,
               wsk_ref, out_ref):
    h = jnp.dot(nf_ref[...], w1_ref[...], preferred_element_type=jnp.float32)
    s = jnp.dot(p0_ref[...] + p1_ref[...], wm4_ref[...],
                preferred_element_type=jnp.float32)
    o = jnp.dot(h * s, w2_ref[...], preferred_element_type=jnp.float32)
    o = o * AVG_INV
    attr = attr_ref[...]
    acc = attr[:, 0:1] * jnp.dot(o, wsk_ref[0:D, :],
                                 preferred_element_type=jnp.float32)
    for j in range(1, ZA):
        acc = acc + attr[:, j:j + 1] * jnp.dot(
            o, wsk_ref[j * D:(j + 1) * D, :],
            preferred_element_type=jnp.float32)
    out_ref[...] = acc


def _node_stage(node_feat, p0, p1, node_attr, W1, Wm4, W2, wsk2):
    BN = 1000
    return pl.pallas_call(
        _node_body,
        grid=(N // BN,),
        in_specs=[
            pl.BlockSpec((BN, D), lambda i: (i, 0)),
            pl.BlockSpec((BN, HID), lambda i: (i, 0)),
            pl.BlockSpec((BN, HID), lambda i: (i, 0)),
            pl.BlockSpec((BN, ZA), lambda i: (i, 0)),
            pl.BlockSpec((D, D), lambda i: (0, 0)),
            pl.BlockSpec((HID, D), lambda i: (0, 0)),
            pl.BlockSpec((D, D), lambda i: (0, 0)),
            pl.BlockSpec((ZA * D, D), lambda i: (0, 0)),
        ],
        out_specs=pl.BlockSpec((BN, D), lambda i: (i, 0)),
        out_shape=jax.ShapeDtypeStruct((N, D), jnp.float32),
    )(node_feat, p0, p1, node_attr, W1, Wm4, W2, wsk2)


def kernel(node_feat, edge_idx, edge_diff_embedding, edge_dist_embedding,
           node_attr, W1, Wm1, Wm2, Wm3, Wm4, W2, Wskip):
    pad = EPAD - E
    dist_pad = jnp.pad(edge_dist_embedding, ((0, pad), (0, 0)))
    diff_pad = jnp.pad(edge_diff_embedding, ((0, pad), (0, 0)))
    src_pad = jnp.pad(edge_idx[:, 0], (0, pad)).reshape(NW * CPT, CHUNK)

    v = _edge_stage(dist_pad, diff_pad, Wm1, Wm2, Wm3)

    zeros = jnp.zeros((NPS, HID), jnp.float32)
    parts = _get_sc_scatter()(v, src_pad, zeros)
    p0 = parts[:N]
    p1 = parts[NPAD:NPAD + N]

    wsk2 = jnp.transpose(Wskip, (1, 0, 2)).reshape(ZA * D, D)
    out = _node_stage(node_feat, p0, p1, node_attr, W1, Wm4, W2, wsk2)
    return out.reshape(N, D, 1)


# speedup vs baseline: 2.5103x; 1.1070x over previous
import functools

import jax
import jax.numpy as jnp
from jax import lax
from jax.experimental import pallas as pl
from jax.experimental.pallas import tpu as pltpu
from jax.experimental.pallas import tpu_sc as plsc

N = 10000
E = 320000
D = 128
RB = 8
ZA = 4
HID = 64
AVG_INV = 1.0 / 32.0

NC = 2
NS = 16
NW = NC * NS
CHUNK = 128
CPT = 80
EPAD = CPT * CHUNK * NW
NPAD = 10240
NPS = NPAD // NS


def _edge_body(dist_ref, diff_ref, wm1_ref, wm2_ref, wm3_ref, v_ref):
    x = jnp.dot(dist_ref[...], wm1_ref[...], preferred_element_type=jnp.float32)
    x = x * jax.nn.sigmoid(x)
    x = jnp.dot(x, wm2_ref[...], preferred_element_type=jnp.float32)
    x = x * jax.nn.sigmoid(x)
    x = jnp.dot(x, wm3_ref[...], preferred_element_type=jnp.float32)
    x = x * jax.nn.sigmoid(x)
    v_ref[...] = x * diff_ref[...]


def _edge_stage(dist_pad, diff_pad, Wm1, Wm2, Wm3):
    BE = 2048
    return pl.pallas_call(
        _edge_body,
        grid=(EPAD // BE,),
        in_specs=[
            pl.BlockSpec((BE, RB), lambda i: (i, 0)),
            pl.BlockSpec((BE, 1), lambda i: (i, 0)),
            pl.BlockSpec((RB, HID), lambda i: (0, 0)),
            pl.BlockSpec((HID, HID), lambda i: (0, 0)),
            pl.BlockSpec((HID, HID), lambda i: (0, 0)),
        ],
        out_specs=pl.BlockSpec((BE, HID), lambda i: (i, 0)),
        out_shape=jax.ShapeDtypeStruct((EPAD, HID), jnp.float32),
    )(dist_pad, diff_pad, Wm1, Wm2, Wm3)


def _sc_body(v_hbm, idx_hbm, zeros_hbm, out_hbm, shared,
             idx0, idx1, vb0, vb1, si0, si1, sv0, sv1):
    cid = lax.axis_index("c")
    sid = lax.axis_index("s")
    wid = sid * NC + cid
    base = wid * CPT
    idxb = (idx0, idx1)
    vb = (vb0, vb1)
    sem_i = (si0, si1)
    sem_v = (sv0, sv1)

    def start(row, b):
        pltpu.async_copy(idx_hbm.at[row], idxb[b], sem_i[b])
        pltpu.async_copy(v_hbm.at[pl.ds(row * CHUNK, CHUNK)], vb[b], sem_v[b])

    def drain(row, b):
        pltpu.make_async_copy(idx_hbm.at[row], idxb[b], sem_i[b]).wait()
        pltpu.make_async_copy(v_hbm.at[pl.ds(row * CHUNK, CHUNK)], vb[b],
                              sem_v[b]).wait()

    start(base, 0)
    pltpu.sync_copy(zeros_hbm, shared.at[pl.ds(sid * NPS, NPS)])
    plsc.subcore_barrier()

    def step(k, carry):
        start(base + 2 * k + 1, 1)
        drain(base + 2 * k, 0)
        pltpu.sync_copy(vb0, shared.at[idx0], add=True)

        @pl.when(k < CPT // 2 - 1)
        def _():
            start(base + 2 * k + 2, 0)
        drain(base + 2 * k + 1, 1)
        pltpu.sync_copy(vb1, shared.at[idx1], add=True)
        return carry

    lax.fori_loop(0, CPT // 2, step, 0)
    plsc.subcore_barrier()
    out_base = cid * NPAD + sid * NPS
    pltpu.sync_copy(shared.at[pl.ds(sid * NPS, NPS)],
                    out_hbm.at[pl.ds(out_base, NPS)])


@functools.cache
def _get_sc_scatter():
    return pl.kernel(
        _sc_body,
        out_type=jax.ShapeDtypeStruct((NC * NPAD, HID), jnp.float32),
        mesh=plsc.VectorSubcoreMesh(core_axis_name="c", subcore_axis_name="s",
                                    num_cores=NC, num_subcores=NS),
        scratch_types=[
            pltpu.VMEM_SHARED((NPAD, HID), jnp.float32),
            pltpu.VMEM((CHUNK,), jnp.int32),
            pltpu.VMEM((CHUNK,), jnp.int32),
            pltpu.VMEM((CHUNK, HID), jnp.float32),
            pltpu.VMEM((CHUNK, HID), jnp.float32),
            pltpu.SemaphoreType.DMA,
            pltpu.SemaphoreType.DMA,
            pltpu.SemaphoreType.DMA,
            pltpu.SemaphoreType.DMA,
        ],
        compiler_params=pltpu.CompilerParams(use_tc_tiling_on_sc=False),
    )


def _node_body(nf_ref, p0_ref, p1_ref, attr_ref, w1_ref, wm4_ref, w2_ref,
               wsk_ref, out_ref):
    h = jnp.dot(nf_ref[...], w1_ref[...], preferred_element_type=jnp.float32)
    s = jnp.dot(p0_ref[...] + p1_ref[...], wm4_ref[...],
                preferred_element_type=jnp.float32)
    o = jnp.dot(h * s, w2_ref[...], preferred_element_type=jnp.float32)
    o = o * AVG_INV
    attr = attr_ref[...]
    acc = attr[:, 0:1] * jnp.dot(o, wsk_ref[0:D, :],
                                 preferred_element_type=jnp.float32)
    for j in range(1, ZA):
        acc = acc + attr[:, j:j + 1] * jnp.dot(
            o, wsk_ref[j * D:(j + 1) * D, :],
            preferred_element_type=jnp.float32)
    out_ref[...] = acc


def _node_stage(node_feat, p0, p1, node_attr, W1, Wm4, W2, wsk2):
    BN = 1000
    return pl.pallas_call(
        _node_body,
        grid=(N // BN,),
        in_specs=[
            pl.BlockSpec((BN, D), lambda i: (i, 0)),
            pl.BlockSpec((BN, HID), lambda i: (i, 0)),
            pl.BlockSpec((BN, HID), lambda i: (i, 0)),
            pl.BlockSpec((BN, ZA), lambda i: (i, 0)),
            pl.BlockSpec((D, D), lambda i: (0, 0)),
            pl.BlockSpec((HID, D), lambda i: (0, 0)),
            pl.BlockSpec((D, D), lambda i: (0, 0)),
            pl.BlockSpec((ZA * D, D), lambda i: (0, 0)),
        ],
        out_specs=pl.BlockSpec((BN, D), lambda i: (i, 0)),
        out_shape=jax.ShapeDtypeStruct((N, D), jnp.float32),
    )(node_feat, p0, p1, node_attr, W1, Wm4, W2, wsk2)


def kernel(node_feat, edge_idx, edge_diff_embedding, edge_dist_embedding,
           node_attr, W1, Wm1, Wm2, Wm3, Wm4, W2, Wskip):
    pad = EPAD - E
    dist_pad = jnp.pad(edge_dist_embedding, ((0, pad), (0, 0)))
    diff_pad = jnp.pad(edge_diff_embedding, ((0, pad), (0, 0)))
    src_pad = jnp.pad(edge_idx[:, 0], (0, pad)).reshape(NW * CPT, CHUNK)

    v = _edge_stage(dist_pad, diff_pad, Wm1, Wm2, Wm3)

    zeros = jnp.zeros((NPS, HID), jnp.float32)
    parts = _get_sc_scatter()(v, src_pad, zeros)
    p0 = parts[:N]
    p1 = parts[NPAD:NPAD + N]

    wsk2 = jnp.transpose(Wskip, (1, 0, 2)).reshape(ZA * D, D)
    out = _node_stage(node_feat, p0, p1, node_attr, W1, Wm4, W2, wsk2)
    return out.reshape(N, D, 1)


# speedup vs baseline: 3.0490x; 1.2146x over previous
import functools

import jax
import jax.numpy as jnp
from jax import lax
from jax.experimental import pallas as pl
from jax.experimental.pallas import tpu as pltpu
from jax.experimental.pallas import tpu_sc as plsc

N = 10000
E = 320000
D = 128
RB = 8
ZA = 4
HID = 64
AVG_INV = 1.0 / 32.0

NC = 2
NS = 16
NW = NC * NS
CHUNK = 128
CPT = 80
EPAD = CPT * CHUNK * NW
NPAD = 10240
NPS = NPAD // NS


_BE = 2048
_LASTB = (E - 1) // _BE


def _edge_body(dist_ref, diff_ref, wm1_ref, wm2_ref, wm3_ref, v_ref):
    i = pl.program_id(0)
    x = jnp.dot(dist_ref[...], wm1_ref[...], preferred_element_type=jnp.float32)
    x = x * jax.nn.sigmoid(x)
    x = jnp.dot(x, wm2_ref[...], preferred_element_type=jnp.float32)
    x = x * jax.nn.sigmoid(x)
    x = jnp.dot(x, wm3_ref[...], preferred_element_type=jnp.float32)
    x = x * jax.nn.sigmoid(x)
    x = x * diff_ref[...]
    rows = i * _BE + jax.lax.broadcasted_iota(jnp.int32, (_BE, 1), 0)
    v_ref[...] = jnp.where(rows < E, x, 0.0)


def _edge_stage(dist, diff, Wm1, Wm2, Wm3):
    clamp = lambda i: (jnp.minimum(i, _LASTB), 0)
    return pl.pallas_call(
        _edge_body,
        grid=(EPAD // _BE,),
        in_specs=[
            pl.BlockSpec((_BE, RB), clamp),
            pl.BlockSpec((_BE, 1), clamp),
            pl.BlockSpec((RB, HID), lambda i: (0, 0)),
            pl.BlockSpec((HID, HID), lambda i: (0, 0)),
            pl.BlockSpec((HID, HID), lambda i: (0, 0)),
        ],
        out_specs=pl.BlockSpec((_BE, HID), lambda i: (i, 0)),
        out_shape=jax.ShapeDtypeStruct((EPAD, HID), jnp.float32),
    )(dist, diff, Wm1, Wm2, Wm3)


def _sc_body(v_hbm, idx_hbm, zeros_hbm, out_hbm, shared,
             idx0, idx1, vb0, vb1, si0, si1, sv0, sv1):
    cid = lax.axis_index("c")
    sid = lax.axis_index("s")
    wid = sid * NC + cid
    base = wid * CPT
    idxb = (idx0, idx1)
    vb = (vb0, vb1)
    sem_i = (si0, si1)
    sem_v = (sv0, sv1)

    def start(row, b):
        pltpu.async_copy(idx_hbm.at[row], idxb[b], sem_i[b])
        pltpu.async_copy(v_hbm.at[pl.ds(row * CHUNK, CHUNK)], vb[b], sem_v[b])

    def drain(row, b):
        pltpu.make_async_copy(idx_hbm.at[row], idxb[b], sem_i[b]).wait()
        pltpu.make_async_copy(v_hbm.at[pl.ds(row * CHUNK, CHUNK)], vb[b],
                              sem_v[b]).wait()

    start(base, 0)
    pltpu.sync_copy(zeros_hbm, shared.at[pl.ds(sid * NPS, NPS)])
    plsc.subcore_barrier()

    def step(k, carry):
        start(base + 2 * k + 1, 1)
        drain(base + 2 * k, 0)
        pltpu.sync_copy(vb0, shared.at[idx0], add=True)

        @pl.when(k < CPT // 2 - 1)
        def _():
            start(base + 2 * k + 2, 0)
        drain(base + 2 * k + 1, 1)
        pltpu.sync_copy(vb1, shared.at[idx1], add=True)
        return carry

    lax.fori_loop(0, CPT // 2, step, 0)
    plsc.subcore_barrier()
    out_base = cid * NPAD + sid * NPS
    pltpu.sync_copy(shared.at[pl.ds(sid * NPS, NPS)],
                    out_hbm.at[pl.ds(out_base, NPS)])


@functools.cache
def _get_sc_scatter():
    return pl.kernel(
        _sc_body,
        out_type=jax.ShapeDtypeStruct((NC * NPAD, HID), jnp.float32),
        mesh=plsc.VectorSubcoreMesh(core_axis_name="c", subcore_axis_name="s",
                                    num_cores=NC, num_subcores=NS),
        scratch_types=[
            pltpu.VMEM_SHARED((NPAD, HID), jnp.float32),
            pltpu.VMEM((CHUNK,), jnp.int32),
            pltpu.VMEM((CHUNK,), jnp.int32),
            pltpu.VMEM((CHUNK, HID), jnp.float32),
            pltpu.VMEM((CHUNK, HID), jnp.float32),
            pltpu.SemaphoreType.DMA,
            pltpu.SemaphoreType.DMA,
            pltpu.SemaphoreType.DMA,
            pltpu.SemaphoreType.DMA,
        ],
        compiler_params=pltpu.CompilerParams(use_tc_tiling_on_sc=False),
    )


def _node_body(nf_ref, p0_ref, p1_ref, attr_ref, w1_ref, wm4_ref, w2_ref,
               wsk_ref, out_ref):
    h = jnp.dot(nf_ref[...], w1_ref[...], preferred_element_type=jnp.float32)
    s = jnp.dot(p0_ref[...] + p1_ref[...], wm4_ref[...],
                preferred_element_type=jnp.float32)
    o = jnp.dot(h * s, w2_ref[...], preferred_element_type=jnp.float32)
    o = o * AVG_INV
    attr = attr_ref[...]
    acc = attr[:, 0:1] * jnp.dot(o, wsk_ref[0:D, :],
                                 preferred_element_type=jnp.float32)
    for j in range(1, ZA):
        acc = acc + attr[:, j:j + 1] * jnp.dot(
            o, wsk_ref[j * D:(j + 1) * D, :],
            preferred_element_type=jnp.float32)
    out_ref[...] = acc


def _node_stage(node_feat, p0, p1, node_attr, W1, Wm4, W2, wsk2):
    BN = 1000
    return pl.pallas_call(
        _node_body,
        grid=(N // BN,),
        in_specs=[
            pl.BlockSpec((BN, D), lambda i: (i, 0)),
            pl.BlockSpec((BN, HID), lambda i: (i, 0)),
            pl.BlockSpec((BN, HID), lambda i: (i, 0)),
            pl.BlockSpec((BN, ZA), lambda i: (i, 0)),
            pl.BlockSpec((D, D), lambda i: (0, 0)),
            pl.BlockSpec((HID, D), lambda i: (0, 0)),
            pl.BlockSpec((D, D), lambda i: (0, 0)),
            pl.BlockSpec((ZA * D, D), lambda i: (0, 0)),
        ],
        out_specs=pl.BlockSpec((BN, D), lambda i: (i, 0)),
        out_shape=jax.ShapeDtypeStruct((N, D), jnp.float32),
    )(node_feat, p0, p1, node_attr, W1, Wm4, W2, wsk2)


def kernel(node_feat, edge_idx, edge_diff_embedding, edge_dist_embedding,
           node_attr, W1, Wm1, Wm2, Wm3, Wm4, W2, Wskip):
    pad = EPAD - E
    src_pad = jnp.pad(edge_idx[:, 0], (0, pad)).reshape(NW * CPT, CHUNK)

    v = _edge_stage(edge_dist_embedding, edge_diff_embedding, Wm1, Wm2, Wm3)

    zeros = jnp.zeros((NPS, HID), jnp.float32)
    parts = _get_sc_scatter()(v, src_pad, zeros)
    p0 = parts[:N]
    p1 = parts[NPAD:NPAD + N]

    wsk2 = jnp.transpose(Wskip, (1, 0, 2)).reshape(ZA * D, D)
    out = _node_stage(node_feat, p0, p1, node_attr, W1, Wm4, W2, wsk2)
    return out.reshape(N, D, 1)
